# Initial kernel scaffold; baseline (speedup 1.0000x reference)
import jax
import jax.numpy as jnp
from jax.experimental import pallas as pl


def kernel(pred, target):
    raise NotImplementedError("write your pallas kernel here")


# speedup vs baseline: 1.0369x; 1.0369x over previous
import functools

import jax
import jax.numpy as jnp
from jax.experimental import pallas as pl

_ALPHA = 0.5
_MARGIN = 0.1
_NUM_SAMPLES = 5000
_LANES = 128
_BLOCK_ROWS = 2048
_PAD_ROWS = 40


def _rank_reduce_kernel(t_ref, o_ref, s_ref, n_ref):
    i = pl.program_id(0)

    @pl.when(i == 0)
    def _init():
        s_ref[...] = jnp.zeros_like(s_ref)
        n_ref[...] = jnp.zeros_like(n_ref)

    t = (t_ref[...] == 1).astype(jnp.float32)
    r = o_ref[...].astype(jnp.float32) + 1.0
    s_ref[...] = s_ref[...] + jnp.sum(t * r)
    n_ref[...] = n_ref[...] + jnp.sum(t)


def _pair_loss_kernel(s0_ref, s1_ref, v_ref, ssum_ref, npos_ref, o_ref, *, length):
    v = v_ref[...]
    diff = s0_ref[...] - s1_ref[...]
    hit = jnp.where((diff < _MARGIN) & (v > 0.0), 1.0, 0.0).astype(jnp.float32)
    vp = jnp.sum(hit)
    cnt = jnp.sum(v)
    l_sort = jnp.where(cnt > 0.0, vp / jnp.maximum(cnt, 1.0), jnp.float32(0.0))
    npos = jnp.sum(npos_ref[...])
    l_rank = jnp.sum(ssum_ref[...]) / jnp.maximum(npos, 1.0) / jnp.float32(length)
    out = l_rank + jnp.float32(_ALPHA) * l_sort
    o_ref[...] = jnp.where(npos < 2.0, jnp.zeros_like(o_ref), jnp.full_like(o_ref[...], out))


def kernel(pred, target):
    length = pred.size
    pred_flat = pred.reshape(-1)
    pred_s = jax.nn.sigmoid(pred_flat).astype(jnp.float32)
    tgt_flat = target.reshape(-1)
    pos_mask = tgt_flat == 1

    order = jnp.argsort(-pred_s)

    rows = length // _LANES
    t2d = tgt_flat.reshape(rows, _LANES)
    o2d = order.reshape(rows, _LANES)
    grid = rows // _BLOCK_ROWS

    ssum, npos = pl.pallas_call(
        _rank_reduce_kernel,
        grid=(grid,),
        in_specs=[
            pl.BlockSpec((_BLOCK_ROWS, _LANES), lambda i: (i, 0)),
            pl.BlockSpec((_BLOCK_ROWS, _LANES), lambda i: (i, 0)),
        ],
        out_specs=[
            pl.BlockSpec((1, 1), lambda i: (0, 0)),
            pl.BlockSpec((1, 1), lambda i: (0, 0)),
        ],
        out_shape=[
            jax.ShapeDtypeStruct((1, 1), jnp.float32),
            jax.ShapeDtypeStruct((1, 1), jnp.float32),
        ],
    )(t2d, o2d)

    num_pos = npos[0, 0].astype(jnp.int32)
    num_pos_safe = jnp.maximum(num_pos, 1)
    pos_order = jnp.argsort(jnp.logical_not(pos_mask))
    idx = jax.random.randint(jax.random.key(42), (2, _NUM_SAMPLES), 0, num_pos_safe)
    s0 = pred_s[pos_order[idx[0]]]
    s1 = pred_s[pos_order[idx[1]]]
    valid = (idx[0] != idx[1]).astype(jnp.float32)

    pad = _PAD_ROWS * _LANES

    def _pad2d(x):
        return (
            jnp.zeros((pad,), jnp.float32)
            .at[:_NUM_SAMPLES]
            .set(x)
            .reshape(_PAD_ROWS, _LANES)
        )

    out = pl.pallas_call(
        functools.partial(_pair_loss_kernel, length=length),
        out_shape=jax.ShapeDtypeStruct((1, 1), jnp.float32),
    )(_pad2d(s0), _pad2d(s1), _pad2d(valid), ssum, npos)
    return out[0, 0]


# speedup vs baseline: 1.5051x; 1.4515x over previous
import functools

import jax
import jax.numpy as jnp
from jax.experimental import pallas as pl

_ALPHA = 0.5
_MARGIN = 0.1
_NUM_SAMPLES = 5000
_LANES = 128
_BLOCK_ROWS = 2048
_PAD_ROWS = 40


def _rank_reduce_kernel(t_ref, o_ref, s_ref, n_ref):
    i = pl.program_id(0)

    @pl.when(i == 0)
    def _init():
        s_ref[...] = jnp.zeros_like(s_ref)
        n_ref[...] = jnp.zeros_like(n_ref)

    t = (t_ref[...] == 1).astype(jnp.float32)
    r = o_ref[...].astype(jnp.float32) + 1.0
    s_ref[...] = s_ref[...] + jnp.sum(t * r)
    n_ref[...] = n_ref[...] + jnp.sum(t)


def _pair_loss_kernel(s0_ref, s1_ref, v_ref, ssum_ref, npos_ref, o_ref, *, length):
    v = v_ref[...]
    diff = s0_ref[...] - s1_ref[...]
    hit = jnp.where((diff < _MARGIN) & (v > 0.0), 1.0, 0.0).astype(jnp.float32)
    vp = jnp.sum(hit)
    cnt = jnp.sum(v)
    l_sort = jnp.where(cnt > 0.0, vp / jnp.maximum(cnt, 1.0), jnp.float32(0.0))
    npos = jnp.sum(npos_ref[...])
    l_rank = jnp.sum(ssum_ref[...]) / jnp.maximum(npos, 1.0) / jnp.float32(length)
    out = l_rank + jnp.float32(_ALPHA) * l_sort
    o_ref[...] = jnp.where(npos < 2.0, jnp.zeros_like(o_ref), jnp.full_like(o_ref[...], out))


def kernel(pred, target):
    length = pred.size
    pred_flat = pred.reshape(-1)
    pred_s = jax.nn.sigmoid(pred_flat).astype(jnp.float32)
    tgt_flat = target.reshape(-1)
    pos_mask = tgt_flat == 1

    order = jnp.argsort(-pred_s)

    rows = length // _LANES
    t2d = tgt_flat.reshape(rows, _LANES)
    o2d = order.reshape(rows, _LANES)
    grid = rows // _BLOCK_ROWS

    ssum, npos = pl.pallas_call(
        _rank_reduce_kernel,
        grid=(grid,),
        in_specs=[
            pl.BlockSpec((_BLOCK_ROWS, _LANES), lambda i: (i, 0)),
            pl.BlockSpec((_BLOCK_ROWS, _LANES), lambda i: (i, 0)),
        ],
        out_specs=[
            pl.BlockSpec((1, 1), lambda i: (0, 0)),
            pl.BlockSpec((1, 1), lambda i: (0, 0)),
        ],
        out_shape=[
            jax.ShapeDtypeStruct((1, 1), jnp.float32),
            jax.ShapeDtypeStruct((1, 1), jnp.float32),
        ],
    )(t2d, o2d)

    num_pos = npos[0, 0].astype(jnp.int32)
    num_pos_safe = jnp.maximum(num_pos, 1)
    idx = jax.random.randint(jax.random.key(42), (2, _NUM_SAMPLES), 0, num_pos_safe)
    csum = jnp.cumsum(pos_mask.astype(jnp.int32))
    g = jnp.clip(jnp.searchsorted(csum, idx + 1, side="left"), 0, length - 1)
    s0 = pred_s[g[0]]
    s1 = pred_s[g[1]]
    valid = (idx[0] != idx[1]).astype(jnp.float32)

    pad = _PAD_ROWS * _LANES

    def _pad2d(x):
        return (
            jnp.zeros((pad,), jnp.float32)
            .at[:_NUM_SAMPLES]
            .set(x)
            .reshape(_PAD_ROWS, _LANES)
        )

    out = pl.pallas_call(
        functools.partial(_pair_loss_kernel, length=length),
        out_shape=jax.ShapeDtypeStruct((1, 1), jnp.float32),
    )(_pad2d(s0), _pad2d(s1), _pad2d(valid), ssum, npos)
    return out[0, 0]
